# Initial kernel scaffold; baseline (speedup 1.0000x reference)
import jax
import jax.numpy as jnp
from jax.experimental import pallas as pl


def kernel(qlen, klen, W):
    raise NotImplementedError("write your pallas kernel here")


# speedup vs baseline: 43.1374x; 43.1374x over previous
import functools
import math

import jax
import jax.numpy as jnp
from jax import lax
from jax.experimental import pallas as pl
from jax.experimental.pallas import tpu as pltpu
from jax.experimental.pallas import tpu_sc as plsc

NUM_BUCKETS = 32
NUM_HEADS = 16
MAX_DISTANCE = 128
Q = 2048
K = 2048
GROUP = 8
DW = 4096
NGROUPS = Q // GROUP
NTEC = 32
GROUPS_PER_TEC = NUM_HEADS * NGROUPS // NTEC
DMA_BATCH = 8


def _table_body(wt_ref, out_ref):
    i = lax.broadcasted_iota(jnp.int32, (GROUP, DW), 0)
    x = lax.broadcasted_iota(jnp.int32, (GROUP, DW), 1)
    n = jnp.maximum(2040 + i - x, 0)
    max_exact = NUM_BUCKETS // 2
    nf = n.astype(jnp.float32)
    val_if_large = max_exact + (
        jnp.log(nf / max_exact + 1e-09)
        / math.log(MAX_DISTANCE / max_exact)
        * (NUM_BUCKETS - max_exact)
    ).astype(jnp.int32)
    val_if_large = jnp.minimum(val_if_large, NUM_BUCKETS - 1)
    bkt = jnp.where(n < max_exact, n, val_if_large)
    wt = wt_ref[...]
    for row in range(GROUP):
        b_iota = lax.broadcasted_iota(jnp.int32, (NUM_BUCKETS, DW), 0)
        onehot = (bkt[row : row + 1, :] == b_iota).astype(jnp.float32)
        out_ref[:, row, :] = jnp.dot(
            wt, onehot, preferred_element_type=jnp.float32
        )


_build_table = pl.pallas_call(
    _table_body,
    out_shape=jax.ShapeDtypeStruct((NUM_HEADS, GROUP, DW), jnp.float32),
)


def _expand_body(dsh_hbm, out_hbm, dsh_v, sem):
    c = lax.axis_index("c")
    s = lax.axis_index("s")
    wid = s * 2 + c
    h = wid // 2
    half = wid - 2 * (wid // 2)
    pltpu.sync_copy(dsh_hbm.at[pl.ds(h * (GROUP * DW), GROUP * DW)], dsh_v)
    g0 = half * (NGROUPS // 2)

    def fire(gi, carry):
        q0 = (g0 + gi) * GROUP
        b = (Q - GROUP) - q0
        for i in range(GROUP):
            pltpu.make_async_copy(
                dsh_v.at[pl.ds(i * DW + b, K)],
                out_hbm.at[pl.ds((h * Q + q0 + i) * K, K)],
                sem,
            ).start()
        return carry

    lax.fori_loop(0, NGROUPS // 2, fire, 0)

    def drain(gi, carry):
        for _ in range(GROUP):
            pltpu.make_async_copy(
                dsh_v.at[pl.ds(0, K)], out_hbm.at[pl.ds(0, K)], sem
            ).wait()
        return carry

    lax.fori_loop(0, NGROUPS // 2, drain, 0)


@functools.cache
def _expand():
    return pl.kernel(
        _expand_body,
        out_type=jax.ShapeDtypeStruct((NUM_HEADS * Q * K,), jnp.float32),
        mesh=plsc.VectorSubcoreMesh(core_axis_name="c", subcore_axis_name="s"),
        scratch_types=[
            pltpu.VMEM((GROUP * DW,), jnp.float32),
            pltpu.SemaphoreType.DMA,
        ],
    )


def kernel(qlen, klen, W):
    wt = W.T
    dsh = _build_table(wt)
    out = _expand()(dsh.reshape(NUM_HEADS * GROUP * DW))
    return out.reshape(1, NUM_HEADS, Q, K)
